# baseline (speedup 1.0000x reference)
import functools

import jax
import jax.numpy as jnp
import numpy as np
from jax import lax
from jax.experimental import pallas as pl
from jax.experimental.pallas import tpu as pltpu
from jax.experimental.pallas import tpu_sc as plsc

B = 25000
D = 128
S = 20
NV = D // 16
NS = 16
BPW = 1568
B_PAD = NS * BPW
Q = 16
NCH = BPW // Q
G = 4
GROWS = Q * S // G

LAST_VALID = B - (NS - 1) * BPW
LAST_VALID_N = LAST_VALID * S

_PERM = np.zeros((D,), np.int32)
for _v in range(D // 32):
    for _k in range(16):
        _PERM[32 * _v + _k] = 32 * _v + 2 * _k
        _PERM[32 * _v + 16 + _k] = 32 * _v + 2 * _k + 1

_mesh = plsc.VectorSubcoreMesh(core_axis_name="c", subcore_axis_name="s")


@functools.partial(
    pl.kernel,
    out_type=jax.ShapeDtypeStruct((B_PAD, 2 * D), jnp.float32),
    mesh=_mesh,
    compiler_params=pltpu.CompilerParams(use_tc_tiling_on_sc=False),
    scratch_types=[
        pltpu.VMEM((BPW,), jnp.int32),
        pltpu.VMEM((BPW * S,), jnp.int32),
        pltpu.VMEM((Q * S, D // 2), jnp.int32),
        pltpu.VMEM((Q * S, D // 2), jnp.int32),
        pltpu.VMEM((Q, D), jnp.float32),
        pltpu.VMEM((Q, D), jnp.float32),
        pltpu.VMEM((Q, 2 * D), jnp.float32),
        pltpu.VMEM((Q, 2 * D), jnp.float32),
        pltpu.SemaphoreType.DMA,
        pltpu.SemaphoreType.DMA,
        pltpu.SemaphoreType.DMA,
        pltpu.SemaphoreType.DMA,
    ],
)
def _sc_gather(feat_hbm, featbf_hbm, nodes_hbm, neigh_hbm, comb_hbm,
               nodes_v, nidx_v, rows0, rows1, srows0, srows1,
               stage0, stage1, gsem0, gsem1, osem0, osem1):
    cid = lax.axis_index("c")
    sid = lax.axis_index("s")

    @pl.when(cid == 0)
    def _run():
        base = sid * BPW

        @pl.when(sid < NS - 1)
        def _load_idx_full():
            pltpu.sync_copy(nodes_hbm.at[pl.ds(base, BPW)], nodes_v)
            pltpu.sync_copy(neigh_hbm.at[pl.ds(base * S, BPW * S)], nidx_v)

        @pl.when(sid == NS - 1)
        def _load_idx_tail():
            zi = jnp.zeros((16,), jnp.int32)

            def zn(i, carry):
                nidx_v[pl.ds(i * 16, 16)] = zi
                return carry

            lax.fori_loop(0, BPW * S // 16, zn, 0)

            def zs(i, carry):
                nodes_v[pl.ds(i * 16, 16)] = zi
                return carry

            lax.fori_loop(0, BPW // 16, zs, 0)
            pltpu.sync_copy(nodes_hbm.at[pl.ds(base, LAST_VALID)],
                            nodes_v.at[pl.ds(0, LAST_VALID)])
            pltpu.sync_copy(neigh_hbm.at[pl.ds(base * S, LAST_VALID_N)],
                            nidx_v.at[pl.ds(0, LAST_VALID_N)])

        rows = (rows0, rows1)
        srows = (srows0, srows1)
        stage = (stage0, stage1)
        gsem = (gsem0, gsem1)
        osem = (osem0, osem1)

        def issue(c, p):
            for g in range(G):
                pltpu.async_copy(
                    featbf_hbm.at[
                        nidx_v.at[pl.ds(c * (Q * S) + g * GROWS, GROWS)]],
                    rows[p].at[pl.ds(g * GROWS, GROWS)],
                    gsem[p])
            pltpu.async_copy(feat_hbm.at[nodes_v.at[pl.ds(c * Q, Q)]],
                             srows[p], gsem[p])

        def wait_gathers(p):
            for g in range(G):
                pltpu.make_async_copy(
                    featbf_hbm.at[pl.ds(0, GROWS)],
                    rows[p].at[pl.ds(g * GROWS, GROWS)],
                    gsem[p]).wait()
            pltpu.make_async_copy(feat_hbm.at[pl.ds(0, Q)], srows[p],
                                  gsem[p]).wait()

        def accum(p):
            r = rows[p]
            sr = srows[p]
            st = stage[p]
            NB = D // 32

            def qbody(q, carry):
                for v in range(NV):
                    sl = pl.ds(v * 16, 16)
                    st[q, sl] = sr[q, sl]
                acce = [None] * NB
                acco = [None] * NB
                for s in range(S):
                    for v in range(NB):
                        xi = r[q * S + s, pl.ds(v * 16, 16)]
                        e = lax.bitcast_convert_type(xi << 16, jnp.float32)
                        o = lax.bitcast_convert_type(
                            xi & jnp.int32(-65536), jnp.float32)
                        if s == 0:
                            acce[v] = e
                            acco[v] = o
                        else:
                            acce[v] = acce[v] + e
                            acco[v] = acco[v] + o
                for v in range(NB):
                    st[q, pl.ds(D + 32 * v, 16)] = acce[v] * (1.0 / S)
                    st[q, pl.ds(D + 32 * v + 16, 16)] = acco[v] * (1.0 / S)
                return carry

            lax.fori_loop(0, Q, qbody, 0)

        issue(0, 0)
        issue(1, 1)

        def tbody(t, carry):
            for p in (0, 1):
                c = 2 * t + p
                wait_gathers(p)

                @pl.when(t > 0)
                def _wait_out():
                    pltpu.make_async_copy(stage[p], comb_hbm.at[pl.ds(0, Q)],
                                          osem[p]).wait()

                accum(p)
                pltpu.async_copy(stage[p],
                                 comb_hbm.at[pl.ds(base + c * Q, Q)],
                                 osem[p])
                cn = c + 2

                @pl.when(cn < NCH)
                def _issue_next():
                    issue(cn, p)
            return carry

        lax.fori_loop(0, NCH // 2, tbody, 0)
        for p in (0, 1):
            pltpu.make_async_copy(stage[p], comb_hbm.at[pl.ds(0, Q)],
                                  osem[p]).wait()


BLK = 1000


def _mm_body(x_ref, w_ref, o_ref):
    o_ref[...] = jnp.maximum(
        jnp.dot(x_ref[...], w_ref[...], preferred_element_type=jnp.float32),
        0.0)


def _tc_matmul(comb, w):
    return pl.pallas_call(
        _mm_body,
        grid=(B // BLK,),
        in_specs=[
            pl.BlockSpec((BLK, 2 * D), lambda i: (i, 0)),
            pl.BlockSpec((2 * D, D), lambda i: (0, 0)),
        ],
        out_specs=pl.BlockSpec((BLK, D), lambda i: (i, 0)),
        out_shape=jax.ShapeDtypeStruct((B, D), jnp.float32),
    )(comb, w)


@jax.jit
def kernel(nodes, neigh_idx, features, weight):
    nodes_i = nodes.astype(jnp.int32)
    neigh_i = neigh_idx.astype(jnp.int32).reshape(-1)
    feat_pk = lax.bitcast_convert_type(
        features.astype(jnp.bfloat16).reshape(-1, D // 2, 2), jnp.int32)
    w2 = jnp.concatenate([weight[:D], weight[D:][_PERM]], axis=0)
    comb = _sc_gather(features, feat_pk, nodes_i, neigh_i)
    return _tc_matmul(comb, w2)


# speedup vs baseline: 2.0250x; 2.0250x over previous
import functools

import jax
import jax.numpy as jnp
from jax import lax
from jax.experimental import pallas as pl
from jax.experimental.pallas import tpu as pltpu
from jax.experimental.pallas import tpu_sc as plsc

B = 25000
D = 128
S = 20
NV = D // 16
NS = 16
BPW = 1568
B_PAD = NS * BPW
Q = 16
NCH = BPW // Q
G = 4
GROWS = Q * S // G

LAST_VALID = B - (NS - 1) * BPW
LAST_VALID_N = LAST_VALID * S

_mesh = plsc.VectorSubcoreMesh(core_axis_name="c", subcore_axis_name="s")


@functools.partial(
    pl.kernel,
    out_type=jax.ShapeDtypeStruct((B_PAD, 2 * D), jnp.float32),
    mesh=_mesh,
    scratch_types=[
        pltpu.VMEM((BPW,), jnp.int32),
        pltpu.VMEM((BPW * S,), jnp.int32),
        pltpu.VMEM((Q * S, D), jnp.float32),
        pltpu.VMEM((Q * S, D), jnp.float32),
        pltpu.VMEM((Q, D), jnp.float32),
        pltpu.VMEM((Q, D), jnp.float32),
        pltpu.VMEM((Q, 2 * D), jnp.float32),
        pltpu.VMEM((Q, 2 * D), jnp.float32),
        pltpu.SemaphoreType.DMA,
        pltpu.SemaphoreType.DMA,
        pltpu.SemaphoreType.DMA,
        pltpu.SemaphoreType.DMA,
    ],
)
def _sc_gather(feat_hbm, nodes_hbm, neigh_hbm, comb_hbm,
               nodes_v, nidx_v, rows0, rows1, srows0, srows1,
               stage0, stage1, gsem0, gsem1, osem0, osem1):
    cid = lax.axis_index("c")
    sid = lax.axis_index("s")

    @pl.when(cid == 0)
    def _run():
        base = sid * BPW

        @pl.when(sid < NS - 1)
        def _load_idx_full():
            pltpu.sync_copy(nodes_hbm.at[pl.ds(base, BPW)], nodes_v)
            pltpu.sync_copy(neigh_hbm.at[pl.ds(base * S, BPW * S)], nidx_v)

        @pl.when(sid == NS - 1)
        def _load_idx_tail():
            zi = jnp.zeros((16,), jnp.int32)

            def zn(i, carry):
                nidx_v[pl.ds(i * 16, 16)] = zi
                return carry

            lax.fori_loop(0, BPW * S // 16, zn, 0)

            def zs(i, carry):
                nodes_v[pl.ds(i * 16, 16)] = zi
                return carry

            lax.fori_loop(0, BPW // 16, zs, 0)
            pltpu.sync_copy(nodes_hbm.at[pl.ds(base, LAST_VALID)],
                            nodes_v.at[pl.ds(0, LAST_VALID)])
            pltpu.sync_copy(neigh_hbm.at[pl.ds(base * S, LAST_VALID_N)],
                            nidx_v.at[pl.ds(0, LAST_VALID_N)])

        rows = (rows0, rows1)
        srows = (srows0, srows1)
        stage = (stage0, stage1)
        gsem = (gsem0, gsem1)
        osem = (osem0, osem1)

        def issue(c, p):
            for g in range(G):
                pltpu.async_copy(
                    feat_hbm.at[
                        nidx_v.at[pl.ds(c * (Q * S) + g * GROWS, GROWS)]],
                    rows[p].at[pl.ds(g * GROWS, GROWS)],
                    gsem[p])
            pltpu.async_copy(feat_hbm.at[nodes_v.at[pl.ds(c * Q, Q)]],
                             srows[p], gsem[p])

        def wait_gathers(p):
            for g in range(G):
                pltpu.make_async_copy(
                    feat_hbm.at[pl.ds(0, GROWS)],
                    rows[p].at[pl.ds(g * GROWS, GROWS)],
                    gsem[p]).wait()
            pltpu.make_async_copy(feat_hbm.at[pl.ds(0, Q)], srows[p],
                                  gsem[p]).wait()

        def accum(p):
            r = rows[p]
            sr = srows[p]
            st = stage[p]

            def qbody(q, carry):
                for v in range(NV):
                    sl = pl.ds(v * 16, 16)
                    st[q, sl] = sr[q, sl]
                accs = [r[q * S, pl.ds(v * 16, 16)] for v in range(NV)]
                for s in range(1, S):
                    for v in range(NV):
                        accs[v] = accs[v] + r[q * S + s, pl.ds(v * 16, 16)]
                for v in range(NV):
                    st[q, pl.ds(D + v * 16, 16)] = accs[v] * (1.0 / S)
                return carry

            lax.fori_loop(0, Q, qbody, 0)

        issue(0, 0)
        issue(1, 1)

        def tbody(t, carry):
            for p in (0, 1):
                c = 2 * t + p
                wait_gathers(p)

                @pl.when(t > 0)
                def _wait_out():
                    pltpu.make_async_copy(stage[p], comb_hbm.at[pl.ds(0, Q)],
                                          osem[p]).wait()

                accum(p)
                pltpu.async_copy(stage[p],
                                 comb_hbm.at[pl.ds(base + c * Q, Q)],
                                 osem[p])
                cn = c + 2

                @pl.when(cn < NCH)
                def _issue_next():
                    issue(cn, p)
            return carry

        lax.fori_loop(0, NCH // 2, tbody, 0)
        for p in (0, 1):
            pltpu.make_async_copy(stage[p], comb_hbm.at[pl.ds(0, Q)],
                                  osem[p]).wait()


BLK = 1000


def _mm_body(x_ref, w_ref, o_ref):
    o_ref[...] = jnp.maximum(
        jnp.dot(x_ref[...], w_ref[...], preferred_element_type=jnp.float32),
        0.0)


def _tc_matmul(comb, w):
    return pl.pallas_call(
        _mm_body,
        grid=(B // BLK,),
        in_specs=[
            pl.BlockSpec((BLK, 2 * D), lambda i: (i, 0)),
            pl.BlockSpec((2 * D, D), lambda i: (0, 0)),
        ],
        out_specs=pl.BlockSpec((BLK, D), lambda i: (i, 0)),
        out_shape=jax.ShapeDtypeStruct((B, D), jnp.float32),
    )(comb, w)


@jax.jit
def kernel(nodes, neigh_idx, features, weight):
    nodes_i = nodes.astype(jnp.int32)
    neigh_i = neigh_idx.astype(jnp.int32).reshape(-1)
    comb = _sc_gather(features, nodes_i, neigh_i)
    return _tc_matmul(comb, weight)


# speedup vs baseline: 2.0774x; 1.0259x over previous
import functools

import jax
import jax.numpy as jnp
from jax import lax
from jax.experimental import pallas as pl
from jax.experimental.pallas import tpu as pltpu
from jax.experimental.pallas import tpu_sc as plsc

B = 25000
D = 128
S = 20
NV = D // 16
NS = 16
BPW = 1568
B_PAD = NS * BPW
Q = 16
NCH = BPW // Q
G = 1
GROWS = Q * S // G

LAST_VALID = B - (NS - 1) * BPW
LAST_VALID_N = LAST_VALID * S

_mesh = plsc.VectorSubcoreMesh(core_axis_name="c", subcore_axis_name="s")


@functools.partial(
    pl.kernel,
    out_type=jax.ShapeDtypeStruct((B_PAD, 2 * D), jnp.float32),
    mesh=_mesh,
    scratch_types=[
        pltpu.VMEM((BPW,), jnp.int32),
        pltpu.VMEM((BPW * S,), jnp.int32),
        pltpu.VMEM((Q * S, D), jnp.float32),
        pltpu.VMEM((Q * S, D), jnp.float32),
        pltpu.VMEM((Q, D), jnp.float32),
        pltpu.VMEM((Q, D), jnp.float32),
        pltpu.VMEM((Q, 2 * D), jnp.float32),
        pltpu.VMEM((Q, 2 * D), jnp.float32),
        pltpu.SemaphoreType.DMA,
        pltpu.SemaphoreType.DMA,
        pltpu.SemaphoreType.DMA,
        pltpu.SemaphoreType.DMA,
    ],
)
def _sc_gather(feat_hbm, nodes_hbm, neigh_hbm, comb_hbm,
               nodes_v, nidx_v, rows0, rows1, srows0, srows1,
               stage0, stage1, gsem0, gsem1, osem0, osem1):
    cid = lax.axis_index("c")
    sid = lax.axis_index("s")

    @pl.when(cid == 0)
    def _run():
        base = sid * BPW

        @pl.when(sid < NS - 1)
        def _load_idx_full():
            pltpu.sync_copy(nodes_hbm.at[pl.ds(base, BPW)], nodes_v)
            pltpu.sync_copy(neigh_hbm.at[pl.ds(base * S, BPW * S)], nidx_v)

        @pl.when(sid == NS - 1)
        def _load_idx_tail():
            zi = jnp.zeros((16,), jnp.int32)

            def zn(i, carry):
                nidx_v[pl.ds(i * 16, 16)] = zi
                return carry

            lax.fori_loop(0, BPW * S // 16, zn, 0)

            def zs(i, carry):
                nodes_v[pl.ds(i * 16, 16)] = zi
                return carry

            lax.fori_loop(0, BPW // 16, zs, 0)
            pltpu.sync_copy(nodes_hbm.at[pl.ds(base, LAST_VALID)],
                            nodes_v.at[pl.ds(0, LAST_VALID)])
            pltpu.sync_copy(neigh_hbm.at[pl.ds(base * S, LAST_VALID_N)],
                            nidx_v.at[pl.ds(0, LAST_VALID_N)])

        rows = (rows0, rows1)
        srows = (srows0, srows1)
        stage = (stage0, stage1)
        gsem = (gsem0, gsem1)
        osem = (osem0, osem1)

        def issue(c, p):
            for g in range(G):
                pltpu.async_copy(
                    feat_hbm.at[
                        nidx_v.at[pl.ds(c * (Q * S) + g * GROWS, GROWS)]],
                    rows[p].at[pl.ds(g * GROWS, GROWS)],
                    gsem[p])
            pltpu.async_copy(feat_hbm.at[nodes_v.at[pl.ds(c * Q, Q)]],
                             srows[p], gsem[p])

        def wait_gathers(p):
            for g in range(G):
                pltpu.make_async_copy(
                    feat_hbm.at[pl.ds(0, GROWS)],
                    rows[p].at[pl.ds(g * GROWS, GROWS)],
                    gsem[p]).wait()
            pltpu.make_async_copy(feat_hbm.at[pl.ds(0, Q)], srows[p],
                                  gsem[p]).wait()

        def accum(p):
            r = rows[p]
            sr = srows[p]
            st = stage[p]

            def qbody(q, carry):
                for v in range(NV):
                    sl = pl.ds(v * 16, 16)
                    st[q, sl] = sr[q, sl]
                accs = [r[q * S, pl.ds(v * 16, 16)] for v in range(NV)]
                for s in range(1, S):
                    for v in range(NV):
                        accs[v] = accs[v] + r[q * S + s, pl.ds(v * 16, 16)]
                for v in range(NV):
                    st[q, pl.ds(D + v * 16, 16)] = accs[v] * (1.0 / S)
                return carry

            lax.fori_loop(0, Q, qbody, 0)

        issue(0, 0)
        issue(1, 1)

        def tbody(t, carry):
            for p in (0, 1):
                c = 2 * t + p
                wait_gathers(p)

                @pl.when(t > 0)
                def _wait_out():
                    pltpu.make_async_copy(stage[p], comb_hbm.at[pl.ds(0, Q)],
                                          osem[p]).wait()

                accum(p)
                pltpu.async_copy(stage[p],
                                 comb_hbm.at[pl.ds(base + c * Q, Q)],
                                 osem[p])
                cn = c + 2

                @pl.when(cn < NCH)
                def _issue_next():
                    issue(cn, p)
            return carry

        lax.fori_loop(0, NCH // 2, tbody, 0)
        for p in (0, 1):
            pltpu.make_async_copy(stage[p], comb_hbm.at[pl.ds(0, Q)],
                                  osem[p]).wait()


BLK = 5000


def _mm_body(x_ref, w_ref, o_ref):
    o_ref[...] = jnp.maximum(
        jnp.dot(x_ref[...], w_ref[...], preferred_element_type=jnp.float32),
        0.0)


def _tc_matmul(comb, w):
    return pl.pallas_call(
        _mm_body,
        grid=(B // BLK,),
        in_specs=[
            pl.BlockSpec((BLK, 2 * D), lambda i: (i, 0)),
            pl.BlockSpec((2 * D, D), lambda i: (0, 0)),
        ],
        out_specs=pl.BlockSpec((BLK, D), lambda i: (i, 0)),
        out_shape=jax.ShapeDtypeStruct((B, D), jnp.float32),
    )(comb, w)


@jax.jit
def kernel(nodes, neigh_idx, features, weight):
    nodes_i = nodes.astype(jnp.int32)
    neigh_i = neigh_idx.astype(jnp.int32).reshape(-1)
    comb = _sc_gather(features, nodes_i, neigh_i)
    return _tc_matmul(comb, weight)


# speedup vs baseline: 2.1071x; 1.0143x over previous
import functools

import jax
import jax.numpy as jnp
from jax import lax
from jax.experimental import pallas as pl
from jax.experimental.pallas import tpu as pltpu
from jax.experimental.pallas import tpu_sc as plsc

B = 25000
D = 128
S = 20
NV = D // 16
NS = 16
BPW = 1568
B_PAD = NS * BPW
Q = 16
NCH = BPW // Q
G = 1
GROWS = Q * S // G

LAST_VALID = B - (NS - 1) * BPW
LAST_VALID_N = LAST_VALID * S

_mesh = plsc.VectorSubcoreMesh(core_axis_name="c", subcore_axis_name="s")


@functools.partial(
    pl.kernel,
    out_type=jax.ShapeDtypeStruct((B_PAD, D), jnp.int32),
    mesh=_mesh,
    scratch_types=[
        pltpu.VMEM((BPW,), jnp.int32),
        pltpu.VMEM((BPW * S,), jnp.int32),
        pltpu.VMEM((Q * S, D), jnp.float32),
        pltpu.VMEM((Q * S, D), jnp.float32),
        pltpu.VMEM((Q, D), jnp.float32),
        pltpu.VMEM((Q, D), jnp.float32),
        pltpu.VMEM((Q, D), jnp.int32),
        pltpu.VMEM((Q, D), jnp.int32),
        pltpu.SemaphoreType.DMA,
        pltpu.SemaphoreType.DMA,
        pltpu.SemaphoreType.DMA,
        pltpu.SemaphoreType.DMA,
    ],
)
def _sc_gather(feat_hbm, nodes_hbm, neigh_hbm, comb_hbm,
               nodes_v, nidx_v, rows0, rows1, srows0, srows1,
               stage0, stage1, gsem0, gsem1, osem0, osem1):
    cid = lax.axis_index("c")
    sid = lax.axis_index("s")

    @pl.when(cid == 0)
    def _run():
        base = sid * BPW

        @pl.when(sid < NS - 1)
        def _load_idx_full():
            pltpu.sync_copy(nodes_hbm.at[pl.ds(base, BPW)], nodes_v)
            pltpu.sync_copy(neigh_hbm.at[pl.ds(base * S, BPW * S)], nidx_v)

        @pl.when(sid == NS - 1)
        def _load_idx_tail():
            zi = jnp.zeros((16,), jnp.int32)

            def zn(i, carry):
                nidx_v[pl.ds(i * 16, 16)] = zi
                return carry

            lax.fori_loop(0, BPW * S // 16, zn, 0)

            def zs(i, carry):
                nodes_v[pl.ds(i * 16, 16)] = zi
                return carry

            lax.fori_loop(0, BPW // 16, zs, 0)
            pltpu.sync_copy(nodes_hbm.at[pl.ds(base, LAST_VALID)],
                            nodes_v.at[pl.ds(0, LAST_VALID)])
            pltpu.sync_copy(neigh_hbm.at[pl.ds(base * S, LAST_VALID_N)],
                            nidx_v.at[pl.ds(0, LAST_VALID_N)])

        rows = (rows0, rows1)
        srows = (srows0, srows1)
        stage = (stage0, stage1)
        gsem = (gsem0, gsem1)
        osem = (osem0, osem1)

        def issue(c, p):
            for g in range(G):
                pltpu.async_copy(
                    feat_hbm.at[
                        nidx_v.at[pl.ds(c * (Q * S) + g * GROWS, GROWS)]],
                    rows[p].at[pl.ds(g * GROWS, GROWS)],
                    gsem[p])
            pltpu.async_copy(feat_hbm.at[nodes_v.at[pl.ds(c * Q, Q)]],
                             srows[p], gsem[p])

        def wait_gathers(p):
            for g in range(G):
                pltpu.make_async_copy(
                    feat_hbm.at[pl.ds(0, GROWS)],
                    rows[p].at[pl.ds(g * GROWS, GROWS)],
                    gsem[p]).wait()
            pltpu.make_async_copy(feat_hbm.at[pl.ds(0, Q)], srows[p],
                                  gsem[p]).wait()

        def rb16(x):
            xi = lax.bitcast_convert_type(x, jnp.int32)
            rnd = lax.shift_right_logical(xi, 16) & jnp.int32(1)
            return lax.shift_right_logical(xi + jnp.int32(0x7FFF) + rnd, 16)

        def accum(p):
            r = rows[p]
            sr = srows[p]
            st = stage[p]

            def qbody(q, carry):
                accs = [r[q * S, pl.ds(v * 16, 16)] for v in range(NV)]
                for s in range(1, S):
                    for v in range(NV):
                        accs[v] = accs[v] + r[q * S + s, pl.ds(v * 16, 16)]
                for v in range(NV):
                    sl = pl.ds(v * 16, 16)
                    pk = rb16(sr[q, sl]) | (rb16(accs[v] * (1.0 / S)) << 16)
                    st[q, sl] = pk
                return carry

            lax.fori_loop(0, Q, qbody, 0)

        issue(0, 0)
        issue(1, 1)

        def tbody(t, carry):
            for p in (0, 1):
                c = 2 * t + p
                wait_gathers(p)

                @pl.when(t > 0)
                def _wait_out():
                    pltpu.make_async_copy(stage[p], comb_hbm.at[pl.ds(0, Q)],
                                          osem[p]).wait()

                accum(p)
                pltpu.async_copy(stage[p],
                                 comb_hbm.at[pl.ds(base + c * Q, Q)],
                                 osem[p])
                cn = c + 2

                @pl.when(cn < NCH)
                def _issue_next():
                    issue(cn, p)
            return carry

        lax.fori_loop(0, NCH // 2, tbody, 0)
        for p in (0, 1):
            pltpu.make_async_copy(stage[p], comb_hbm.at[pl.ds(0, Q)],
                                  osem[p]).wait()


BLK = 5000


def _mm_body(x_ref, w_ref, o_ref):
    xbf = pltpu.bitcast(x_ref[...], jnp.bfloat16)
    xr = xbf.reshape(BLK, 2, D)
    res = lax.dot_general(
        xr, w_ref[...],
        dimension_numbers=(((2,), (1,)), ((1,), (0,))),
        preferred_element_type=jnp.float32)
    o_ref[...] = jnp.maximum(res[0] + res[1], 0.0)


def _tc_matmul(comb, w2):
    return pl.pallas_call(
        _mm_body,
        grid=(B // BLK,),
        in_specs=[
            pl.BlockSpec((BLK, D), lambda i: (i, 0)),
            pl.BlockSpec((2, D, D), lambda i: (0, 0, 0)),
        ],
        out_specs=pl.BlockSpec((BLK, D), lambda i: (i, 0)),
        out_shape=jax.ShapeDtypeStruct((B, D), jnp.float32),
    )(comb, w2)


@jax.jit
def kernel(nodes, neigh_idx, features, weight):
    nodes_i = nodes.astype(jnp.int32)
    neigh_i = neigh_idx.astype(jnp.int32).reshape(-1)
    w2 = jnp.stack([weight[:D], weight[D:]])
    comb = _sc_gather(features, nodes_i, neigh_i)
    return _tc_matmul(comb, w2)
